# baseline (speedup 1.0000x reference)
import functools

import jax
import jax.numpy as jnp
from jax import lax
from jax.experimental import pallas as pl
from jax.experimental.pallas import tpu as pltpu
from jax.experimental.pallas import tpu_sc as plsc

N, D, F, E = 10000, 256, 256, 160000
NC, NS, L = 2, 16, 16
EP = 163840
CH = 128
ROWS_ALL = EP // CH
NCH_A = EP // (NC * NS) // CH
NCH_B = EP // NS // CH
SROWS = 10240
STRIPE = SROWS // NS
HD = 128
NH = 2
HALF = NCH_B // 2
ZR = 16

_mesh = plsc.VectorSubcoreMesh(core_axis_name="c", subcore_axis_name="s")


def _deg_body(row_hbm, col_hbm, colp_hbm, deg_hbm,
              row_v, col_v, rp_v, ones_v, z_v, deg_sh):
    c = lax.axis_index("c")
    s = lax.axis_index("s")
    w = s * NC + c

    def _fill_ones(i, _):
        ones_v[i, :] = jnp.ones((L,), jnp.float32)
        return 0
    lax.fori_loop(0, CH, _fill_ones, 0)

    def _fill_z(i, _):
        z_v[i, :] = jnp.zeros((L,), jnp.float32)
        return 0
    lax.fori_loop(0, STRIPE, _fill_z, 0)

    pltpu.sync_copy(z_v, deg_sh.at[pl.ds(s * STRIPE, STRIPE)])
    plsc.subcore_barrier()

    base = w * NCH_A
    pltpu.sync_copy(row_hbm.at[pl.ds(base, NCH_A)], row_v)
    pltpu.sync_copy(col_hbm.at[pl.ds(base, NCH_A)], col_v)

    def _remap(i, _):
        j = i // (CH // L)
        g = i % (CH // L)
        r = row_v[j, pl.ds(g * L, L)]
        cc = col_v[j, pl.ds(g * L, L)]
        sl = r == cc
        rp_v[j, pl.ds(g * L, L)] = jnp.where(sl, N, r)
        col_v[j, pl.ds(g * L, L)] = jnp.where(sl, N, cc)
        return 0
    lax.fori_loop(0, NCH_A * (CH // L), _remap, 0)

    def _scat(j, _):
        pltpu.sync_copy(ones_v, deg_sh.at[rp_v.at[j]], add=True)
        return 0
    lax.fori_loop(0, NCH_A, _scat, 0)

    pltpu.sync_copy(col_v, colp_hbm.at[pl.ds(base, NCH_A)])
    plsc.subcore_barrier()

    pltpu.sync_copy(deg_sh.at[pl.ds(s * STRIPE, STRIPE)],
                    deg_hbm.at[pl.ds(s * STRIPE, STRIPE), pl.ds(c * L, L)])


_deg_kernel = functools.partial(
    pl.kernel,
    out_type=(
        jax.ShapeDtypeStruct((ROWS_ALL, CH), jnp.int32),
        jax.ShapeDtypeStruct((SROWS, NC * L), jnp.float32),
    ),
    mesh=_mesh,
    compiler_params=pltpu.CompilerParams(use_tc_tiling_on_sc=False),
    scratch_types=[
        pltpu.VMEM((NCH_A, CH), jnp.int32),
        pltpu.VMEM((NCH_A, CH), jnp.int32),
        pltpu.VMEM((NCH_A, CH), jnp.int32),
        pltpu.VMEM((CH, L), jnp.float32),
        pltpu.VMEM((STRIPE, L), jnp.float32),
        pltpu.VMEM_SHARED((SROWS, L), jnp.float32),
    ],
)(_deg_body)


def _scatter_body(row_hbm, colp_hbm, xtab_hbm, out_hbm,
                  rp_v, cp_v, g0, g1, z_v, S_sh, gs0, gs1, ss0, ss1):
    c = lax.axis_index("c")
    s = lax.axis_index("s")

    def _fill_z(i, _):
        j = i // (HD // L)
        g = i % (HD // L)
        z_v[j, pl.ds(g * L, L)] = jnp.zeros((L,), jnp.float32)
        return 0
    lax.fori_loop(0, ZR * (HD // L), _fill_z, 0)

    def _zero(i, _):
        pltpu.sync_copy(z_v, S_sh.at[pl.ds(s * STRIPE + i * ZR, ZR)])
        return 0
    lax.fori_loop(0, STRIPE // ZR, _zero, 0)

    delta = c * N

    for half in range(2):
        base = s * NCH_B + half * HALF
        pltpu.sync_copy(colp_hbm.at[pl.ds(base, HALF)], cp_v)
        pltpu.sync_copy(row_hbm.at[pl.ds(base, HALF)], rp_v)

        def _addoff(i, _):
            j = i // (CH // L)
            g = i % (CH // L)
            rp_v[j, pl.ds(g * L, L)] = rp_v[j, pl.ds(g * L, L)] + delta
            return 0
        lax.fori_loop(0, HALF * (CH // L), _addoff, 0)
        if half == 0:
            plsc.subcore_barrier()

        pltpu.async_copy(xtab_hbm.at[rp_v.at[0]], g0, gs0)
        pltpu.async_copy(xtab_hbm.at[rp_v.at[1]], g1, gs1)

        def _pipe(h, _):
            j = 2 * h
            pltpu.make_async_copy(xtab_hbm.at[rp_v.at[j]], g0, gs0).wait()
            pltpu.async_copy(g0, S_sh.at[cp_v.at[j]], ss0, add=True)
            pltpu.make_async_copy(
                xtab_hbm.at[rp_v.at[j + 1]], g1, gs1).wait()
            pltpu.async_copy(g1, S_sh.at[cp_v.at[j + 1]], ss1, add=True)
            pltpu.make_async_copy(g0, S_sh.at[cp_v.at[j]], ss0).wait()
            pltpu.async_copy(xtab_hbm.at[rp_v.at[j + 2]], g0, gs0)
            pltpu.make_async_copy(g1, S_sh.at[cp_v.at[j + 1]], ss1).wait()
            pltpu.async_copy(xtab_hbm.at[rp_v.at[j + 3]], g1, gs1)
            return 0
        lax.fori_loop(0, HALF // 2 - 1, _pipe, 0)

        pltpu.make_async_copy(
            xtab_hbm.at[rp_v.at[HALF - 2]], g0, gs0).wait()
        pltpu.async_copy(g0, S_sh.at[cp_v.at[HALF - 2]], ss0, add=True)
        pltpu.make_async_copy(
            xtab_hbm.at[rp_v.at[HALF - 1]], g1, gs1).wait()
        pltpu.async_copy(g1, S_sh.at[cp_v.at[HALF - 1]], ss1, add=True)
        pltpu.make_async_copy(g0, S_sh.at[cp_v.at[HALF - 2]], ss0).wait()
        pltpu.make_async_copy(g1, S_sh.at[cp_v.at[HALF - 1]], ss1).wait()

    plsc.subcore_barrier()
    pltpu.sync_copy(S_sh.at[pl.ds(s * STRIPE, STRIPE)],
                    out_hbm.at[pl.ds(s * STRIPE, STRIPE), pl.ds(c * HD, HD)])


_scatter_kernel = functools.partial(
    pl.kernel,
    out_type=jax.ShapeDtypeStruct((SROWS, NH * HD), jnp.float32),
    mesh=_mesh,
    compiler_params=pltpu.CompilerParams(use_tc_tiling_on_sc=False),
    scratch_types=[
        pltpu.VMEM((HALF, CH), jnp.int32),
        pltpu.VMEM((HALF, CH), jnp.int32),
        pltpu.VMEM((CH, HD), jnp.float32),
        pltpu.VMEM((CH, HD), jnp.float32),
        pltpu.VMEM((ZR, HD), jnp.float32),
        pltpu.VMEM_SHARED((SROWS, HD), jnp.float32),
        pltpu.SemaphoreType.DMA,
        pltpu.SemaphoreType.DMA,
        pltpu.SemaphoreType.DMA,
        pltpu.SemaphoreType.DMA,
    ],
)(_scatter_body)


def _scale_body(x_ref, deg_ref, out_ref):
    deg = deg_ref[:, 0:1] + deg_ref[:, L:L + 1]
    dinv = jnp.where(deg > 0.0, lax.rsqrt(deg), 0.0)
    out_ref[...] = x_ref[...] * dinv


def _scale_x(x, deg2):
    BR = 1000
    grid = N // BR
    return pl.pallas_call(
        _scale_body,
        grid=(NH, grid),
        in_specs=[
            pl.BlockSpec((BR, HD), lambda h, i: (i, h)),
            pl.BlockSpec((BR, NC * L), lambda h, i: (i, 0)),
        ],
        out_specs=pl.BlockSpec((BR, HD), lambda h, i: (h * grid + i, 0)),
        out_shape=jax.ShapeDtypeStruct((NH * N, HD), jnp.float32),
    )(x, deg2)


def _xw_body(x_ref, wz_ref, wh_ref, out_ref):
    out_ref[:, :F] = jnp.dot(x_ref[...], wz_ref[...],
                             preferred_element_type=jnp.float32)
    out_ref[:, F:] = jnp.dot(x_ref[...], wh_ref[...],
                             preferred_element_type=jnp.float32)


def _xw_dense(x, wz0, wh0):
    BR = 2000
    grid = N // BR
    return pl.pallas_call(
        _xw_body,
        grid=(grid,),
        in_specs=[
            pl.BlockSpec((BR, D), lambda i: (i, 0)),
            pl.BlockSpec((D, F), lambda i: (0, 0)),
            pl.BlockSpec((D, F), lambda i: (0, 0)),
        ],
        out_specs=pl.BlockSpec((BR, 2 * F), lambda i: (i, 0)),
        out_shape=jax.ShapeDtypeStruct((N, 2 * F), jnp.float32),
    )(x, wz0, wh0)


def _fused_body(p_ref, s_ref, deg_ref, wz_ref, wh_ref,
                bz_ref, bh_ref, wl_ref, bl_ref, out_ref):
    deg = deg_ref[:, 0:1] + deg_ref[:, L:L + 1]
    ndinv = jnp.where(deg > 0.0, -lax.rsqrt(deg), 0.0)
    t = ndinv * s_ref[...]
    az = (p_ref[:, :F] + bz_ref[...]
          + jnp.dot(t, wz_ref[...], preferred_element_type=jnp.float32))
    ah = (p_ref[:, F:] + bh_ref[...]
          + jnp.dot(t, wh_ref[...], preferred_element_type=jnp.float32))
    h = jnp.maximum((1.0 - jax.nn.sigmoid(az)) * jnp.tanh(ah), 0.0)
    out_ref[...] = (jnp.sum(h * wl_ref[...], axis=1, keepdims=True)
                    + bl_ref[...])


def _fused_dense(p, s2, deg2, wz1, wh1, bz, bh, wlt, bl2):
    BR = 1000
    grid = N // BR
    row = lambda i: (i, 0)
    full = lambda i: (0, 0)
    return pl.pallas_call(
        _fused_body,
        grid=(grid,),
        in_specs=[
            pl.BlockSpec((BR, 2 * F), row),
            pl.BlockSpec((BR, D), row),
            pl.BlockSpec((BR, NC * L), row),
            pl.BlockSpec((D, F), full),
            pl.BlockSpec((D, F), full),
            pl.BlockSpec((1, F), full),
            pl.BlockSpec((1, F), full),
            pl.BlockSpec((1, F), full),
            pl.BlockSpec((1, 1), full),
        ],
        out_specs=pl.BlockSpec((BR, 1), row),
        out_shape=jax.ShapeDtypeStruct((N, 1), jnp.float32),
    )(p, s2, deg2, wz1, wh1, bz, bh, wlt, bl2)


def kernel(x, edge_index, Wxz0, Wxz1, bxz, Whz0, Whz1, bhz, Wxr0, Wxr1, bxr,
           Whr0, Whr1, bhr, Wxh0, Wxh1, bxh, Whh0, Whh1, bhh, Wl, bl):
    row = edge_index[0]
    col = edge_index[1]
    pad = jnp.zeros((EP - E,), jnp.int32)
    row2 = jnp.concatenate([row, pad]).reshape(ROWS_ALL, CH)
    col2 = jnp.concatenate([col, pad]).reshape(ROWS_ALL, CH)

    colp2, deg2 = _deg_kernel(row2, col2)

    xtab = _scale_x(x, deg2)

    p = _xw_dense(x, Wxz0, Wxh0)

    s2 = _scatter_kernel(row2, colp2, xtab)

    return _fused_dense(p, s2, deg2, Wxz1, Wxh1,
                        (bxz + bhz).reshape(1, F), (bxh + bhh).reshape(1, F),
                        Wl.reshape(1, F), bl.reshape(1, 1))


# speedup vs baseline: 1.0563x; 1.0563x over previous
import functools

import jax
import jax.numpy as jnp
from jax import lax
from jax.experimental import pallas as pl
from jax.experimental.pallas import tpu as pltpu
from jax.experimental.pallas import tpu_sc as plsc

N, D, F, E = 10000, 256, 256, 160000
NC, NS, L = 2, 16, 16
EP = 163840
CH = 128
ROWS_ALL = EP // CH
NCH_A = EP // (NC * NS) // CH
NCH_B = EP // NS // CH
SROWS = 10240
STRIPE = SROWS // NS
HD = 128
NH = 2
HALF = NCH_B // 2
ZR = 16

_mesh = plsc.VectorSubcoreMesh(core_axis_name="c", subcore_axis_name="s")


def _deg_body(row_hbm, col_hbm, colp_hbm, deg_hbm,
              row_v, col_v, rp_v, ones_v, z_v, deg_sh):
    c = lax.axis_index("c")
    s = lax.axis_index("s")
    w = s * NC + c

    def _fill_ones(i, _):
        ones_v[i, :] = jnp.ones((L,), jnp.float32)
        return 0
    lax.fori_loop(0, CH, _fill_ones, 0)

    def _fill_z(i, _):
        z_v[i, :] = jnp.zeros((L,), jnp.float32)
        return 0
    lax.fori_loop(0, STRIPE, _fill_z, 0)

    pltpu.sync_copy(z_v, deg_sh.at[pl.ds(s * STRIPE, STRIPE)])
    plsc.subcore_barrier()

    base = w * NCH_A
    pltpu.sync_copy(row_hbm.at[pl.ds(base, NCH_A)], row_v)
    pltpu.sync_copy(col_hbm.at[pl.ds(base, NCH_A)], col_v)

    def _remap(i, _):
        j = i // (CH // L)
        g = i % (CH // L)
        r = row_v[j, pl.ds(g * L, L)]
        cc = col_v[j, pl.ds(g * L, L)]
        sl = r == cc
        rp_v[j, pl.ds(g * L, L)] = jnp.where(sl, N, r)
        col_v[j, pl.ds(g * L, L)] = jnp.where(sl, N, cc)
        return 0
    lax.fori_loop(0, NCH_A * (CH // L), _remap, 0)

    def _scat(j, _):
        pltpu.sync_copy(ones_v, deg_sh.at[rp_v.at[j]], add=True)
        return 0
    lax.fori_loop(0, NCH_A, _scat, 0)

    pltpu.sync_copy(col_v, colp_hbm.at[pl.ds(base, NCH_A)])
    plsc.subcore_barrier()

    pltpu.sync_copy(deg_sh.at[pl.ds(s * STRIPE, STRIPE)],
                    deg_hbm.at[pl.ds(s * STRIPE, STRIPE), pl.ds(c * L, L)])


_deg_kernel = functools.partial(
    pl.kernel,
    out_type=(
        jax.ShapeDtypeStruct((ROWS_ALL, CH), jnp.int32),
        jax.ShapeDtypeStruct((SROWS, NC * L), jnp.float32),
    ),
    mesh=_mesh,
    compiler_params=pltpu.CompilerParams(use_tc_tiling_on_sc=False),
    scratch_types=[
        pltpu.VMEM((NCH_A, CH), jnp.int32),
        pltpu.VMEM((NCH_A, CH), jnp.int32),
        pltpu.VMEM((NCH_A, CH), jnp.int32),
        pltpu.VMEM((CH, L), jnp.float32),
        pltpu.VMEM((STRIPE, L), jnp.float32),
        pltpu.VMEM_SHARED((SROWS, L), jnp.float32),
    ],
)(_deg_body)


def _scatter_body(row_hbm, colp_hbm, xtab_hbm, out_hbm,
                  rp_v, cp_v, g0, g1, z_v, S_sh, gs0, gs1):
    c = lax.axis_index("c")
    s = lax.axis_index("s")

    def _fill_z(i, _):
        j = i // (HD // L)
        g = i % (HD // L)
        z_v[j, pl.ds(g * L, L)] = jnp.zeros((L,), jnp.float32)
        return 0
    lax.fori_loop(0, ZR * (HD // L), _fill_z, 0)

    def _zero(i, _):
        pltpu.sync_copy(z_v, S_sh.at[pl.ds(s * STRIPE + i * ZR, ZR)])
        return 0
    lax.fori_loop(0, STRIPE // ZR, _zero, 0)

    delta = c * N

    for half in range(2):
        base = s * NCH_B + half * HALF
        pltpu.sync_copy(colp_hbm.at[pl.ds(base, HALF)], cp_v)
        pltpu.sync_copy(row_hbm.at[pl.ds(base, HALF)], rp_v)

        def _addoff(i, _):
            j = i // (CH // L)
            g = i % (CH // L)
            rp_v[j, pl.ds(g * L, L)] = rp_v[j, pl.ds(g * L, L)] + delta
            return 0
        lax.fori_loop(0, HALF * (CH // L), _addoff, 0)
        if half == 0:
            plsc.subcore_barrier()

        pltpu.async_copy(xtab_hbm.at[rp_v.at[0]], g0, gs0)
        pltpu.async_copy(xtab_hbm.at[rp_v.at[1]], g1, gs1)

        def _pipe(h, _):
            j = 2 * h
            pltpu.make_async_copy(xtab_hbm.at[rp_v.at[j]], g0, gs0).wait()
            pltpu.sync_copy(g0, S_sh.at[cp_v.at[j]], add=True)
            pltpu.async_copy(xtab_hbm.at[rp_v.at[j + 2]], g0, gs0)
            pltpu.make_async_copy(
                xtab_hbm.at[rp_v.at[j + 1]], g1, gs1).wait()
            pltpu.sync_copy(g1, S_sh.at[cp_v.at[j + 1]], add=True)
            pltpu.async_copy(xtab_hbm.at[rp_v.at[j + 3]], g1, gs1)
            return 0
        lax.fori_loop(0, HALF // 2 - 1, _pipe, 0)

        pltpu.make_async_copy(
            xtab_hbm.at[rp_v.at[HALF - 2]], g0, gs0).wait()
        pltpu.sync_copy(g0, S_sh.at[cp_v.at[HALF - 2]], add=True)
        pltpu.make_async_copy(
            xtab_hbm.at[rp_v.at[HALF - 1]], g1, gs1).wait()
        pltpu.sync_copy(g1, S_sh.at[cp_v.at[HALF - 1]], add=True)

    plsc.subcore_barrier()
    pltpu.sync_copy(S_sh.at[pl.ds(s * STRIPE, STRIPE)],
                    out_hbm.at[pl.ds(c * SROWS + s * STRIPE, STRIPE)])


_scatter_kernel = functools.partial(
    pl.kernel,
    out_type=jax.ShapeDtypeStruct((NH * SROWS, HD), jnp.float32),
    mesh=_mesh,
    compiler_params=pltpu.CompilerParams(use_tc_tiling_on_sc=False),
    scratch_types=[
        pltpu.VMEM((HALF, CH), jnp.int32),
        pltpu.VMEM((HALF, CH), jnp.int32),
        pltpu.VMEM((CH, HD), jnp.float32),
        pltpu.VMEM((CH, HD), jnp.float32),
        pltpu.VMEM((ZR, HD), jnp.float32),
        pltpu.VMEM_SHARED((SROWS, HD), jnp.float32),
        pltpu.SemaphoreType.DMA,
        pltpu.SemaphoreType.DMA,
    ],
)(_scatter_body)


def _scale_body(x_ref, deg_ref, out_ref):
    deg = deg_ref[:, 0:1] + deg_ref[:, L:L + 1]
    dinv = jnp.where(deg > 0.0, lax.rsqrt(deg), 0.0)
    out_ref[...] = x_ref[...] * dinv


def _scale_x(x, deg2):
    BR = 1000
    grid = N // BR
    return pl.pallas_call(
        _scale_body,
        grid=(NH, grid),
        in_specs=[
            pl.BlockSpec((BR, HD), lambda h, i: (i, h)),
            pl.BlockSpec((BR, NC * L), lambda h, i: (i, 0)),
        ],
        out_specs=pl.BlockSpec((BR, HD), lambda h, i: (h * grid + i, 0)),
        out_shape=jax.ShapeDtypeStruct((NH * N, HD), jnp.float32),
    )(x, deg2)


def _xw_body(x_ref, wz_ref, wh_ref, out_ref):
    out_ref[:, :F] = jnp.dot(x_ref[...], wz_ref[...],
                             preferred_element_type=jnp.float32)
    out_ref[:, F:] = jnp.dot(x_ref[...], wh_ref[...],
                             preferred_element_type=jnp.float32)


def _xw_dense(x, wz0, wh0):
    BR = 2000
    grid = N // BR
    return pl.pallas_call(
        _xw_body,
        grid=(grid,),
        in_specs=[
            pl.BlockSpec((BR, D), lambda i: (i, 0)),
            pl.BlockSpec((D, F), lambda i: (0, 0)),
            pl.BlockSpec((D, F), lambda i: (0, 0)),
        ],
        out_specs=pl.BlockSpec((BR, 2 * F), lambda i: (i, 0)),
        out_shape=jax.ShapeDtypeStruct((N, 2 * F), jnp.float32),
    )(x, wz0, wh0)


def _fused_body(p_ref, sa_ref, sb_ref, deg_ref, wz_ref, wh_ref,
                bz_ref, bh_ref, wl_ref, bl_ref, out_ref):
    deg = deg_ref[:, 0:1] + deg_ref[:, L:L + 1]
    ndinv = jnp.where(deg > 0.0, -lax.rsqrt(deg), 0.0)
    t = jnp.concatenate([ndinv * sa_ref[...], ndinv * sb_ref[...]], axis=1)
    az = (p_ref[:, :F] + bz_ref[...]
          + jnp.dot(t, wz_ref[...], preferred_element_type=jnp.float32))
    ah = (p_ref[:, F:] + bh_ref[...]
          + jnp.dot(t, wh_ref[...], preferred_element_type=jnp.float32))
    h = jnp.maximum((1.0 - jax.nn.sigmoid(az)) * jnp.tanh(ah), 0.0)
    out_ref[...] = (jnp.sum(h * wl_ref[...], axis=1, keepdims=True)
                    + bl_ref[...])


def _fused_dense(p, sa, sb, degn, wz1, wh1, bz, bh, wlt, bl2):
    BR = 1000
    grid = N // BR
    row = lambda i: (i, 0)
    full = lambda i: (0, 0)
    return pl.pallas_call(
        _fused_body,
        grid=(grid,),
        in_specs=[
            pl.BlockSpec((BR, 2 * F), row),
            pl.BlockSpec((BR, HD), row),
            pl.BlockSpec((BR, HD), row),
            pl.BlockSpec((BR, NC * L), row),
            pl.BlockSpec((D, F), full),
            pl.BlockSpec((D, F), full),
            pl.BlockSpec((1, F), full),
            pl.BlockSpec((1, F), full),
            pl.BlockSpec((1, F), full),
            pl.BlockSpec((1, 1), full),
        ],
        out_specs=pl.BlockSpec((BR, 1), row),
        out_shape=jax.ShapeDtypeStruct((N, 1), jnp.float32),
    )(p, sa, sb, degn, wz1, wh1, bz, bh, wlt, bl2)


def kernel(x, edge_index, Wxz0, Wxz1, bxz, Whz0, Whz1, bhz, Wxr0, Wxr1, bxr,
           Whr0, Whr1, bhr, Wxh0, Wxh1, bxh, Whh0, Whh1, bhh, Wl, bl):
    row = edge_index[0]
    col = edge_index[1]
    pad = jnp.zeros((EP - E,), jnp.int32)
    row2 = jnp.concatenate([row, pad]).reshape(ROWS_ALL, CH)
    col2 = jnp.concatenate([col, pad]).reshape(ROWS_ALL, CH)

    colp2, deg2 = _deg_kernel(row2, col2)
    degn = lax.slice(deg2, (0, 0), (N, NC * L))

    xtab = _scale_x(x, degn)

    p = _xw_dense(x, Wxz0, Wxh0)

    s2 = _scatter_kernel(row2, colp2, xtab)
    sa = lax.slice(s2, (0, 0), (N, HD))
    sb = lax.slice(s2, (SROWS, 0), (SROWS + N, HD))

    return _fused_dense(p, sa, sb, degn, Wxz1, Wxh1,
                        (bxz + bhz).reshape(1, F), (bxh + bhh).reshape(1, F),
                        Wl.reshape(1, F), bl.reshape(1, 1))


# speedup vs baseline: 1.0687x; 1.0118x over previous
import functools

import jax
import jax.numpy as jnp
from jax import lax
from jax.experimental import pallas as pl
from jax.experimental.pallas import tpu as pltpu
from jax.experimental.pallas import tpu_sc as plsc

N, D, F, E = 10000, 256, 256, 160000
NC, NS, L = 2, 16, 16
EP = 163840
CH = 128
ROWS_ALL = EP // CH
NCH_A = EP // (NC * NS) // CH
NCH_B = EP // NS // CH
SROWS = 10240
STRIPE = SROWS // NS
HD = 128
NH = 2
HALF = NCH_B // 2
ZR = 16

_mesh = plsc.VectorSubcoreMesh(core_axis_name="c", subcore_axis_name="s")


def _deg_body(row_hbm, col_hbm, colp_hbm, deg_hbm,
              row_v, col_v, rp_v, ones_v, z_v, deg_sh):
    c = lax.axis_index("c")
    s = lax.axis_index("s")
    w = s * NC + c

    def _fill_ones(i, _):
        ones_v[i, :] = jnp.ones((L,), jnp.float32)
        return 0
    lax.fori_loop(0, CH, _fill_ones, 0)

    def _fill_z(i, _):
        z_v[i, :] = jnp.zeros((L,), jnp.float32)
        return 0
    lax.fori_loop(0, STRIPE, _fill_z, 0)

    pltpu.sync_copy(z_v, deg_sh.at[pl.ds(s * STRIPE, STRIPE)])
    plsc.subcore_barrier()

    base = w * NCH_A
    pltpu.sync_copy(row_hbm.at[pl.ds(base, NCH_A)], row_v)
    pltpu.sync_copy(col_hbm.at[pl.ds(base, NCH_A)], col_v)

    def _remap(i, _):
        j = i // (CH // L)
        g = i % (CH // L)
        r = row_v[j, pl.ds(g * L, L)]
        cc = col_v[j, pl.ds(g * L, L)]
        sl = r == cc
        rp_v[j, pl.ds(g * L, L)] = jnp.where(sl, N, r)
        col_v[j, pl.ds(g * L, L)] = jnp.where(sl, N, cc)
        return 0
    lax.fori_loop(0, NCH_A * (CH // L), _remap, 0)

    def _scat(j, _):
        pltpu.sync_copy(ones_v, deg_sh.at[rp_v.at[j]], add=True)
        return 0
    lax.fori_loop(0, NCH_A, _scat, 0)

    pltpu.sync_copy(col_v, colp_hbm.at[pl.ds(base, NCH_A)])
    plsc.subcore_barrier()

    pltpu.sync_copy(deg_sh.at[pl.ds(s * STRIPE, STRIPE)],
                    deg_hbm.at[pl.ds(s * STRIPE, STRIPE), pl.ds(c * L, L)])


_deg_kernel = functools.partial(
    pl.kernel,
    out_type=(
        jax.ShapeDtypeStruct((ROWS_ALL, CH), jnp.int32),
        jax.ShapeDtypeStruct((SROWS, NC * L), jnp.float32),
    ),
    mesh=_mesh,
    compiler_params=pltpu.CompilerParams(use_tc_tiling_on_sc=False),
    scratch_types=[
        pltpu.VMEM((NCH_A, CH), jnp.int32),
        pltpu.VMEM((NCH_A, CH), jnp.int32),
        pltpu.VMEM((NCH_A, CH), jnp.int32),
        pltpu.VMEM((CH, L), jnp.float32),
        pltpu.VMEM((STRIPE, L), jnp.float32),
        pltpu.VMEM_SHARED((SROWS, L), jnp.float32),
    ],
)(_deg_body)


CH2 = 64
NCHH = EP // NS // CH2 // 2
NBUF = 4


def _scatter_body(row_hbm, colp_hbm, xtab_hbm, out_hbm,
                  rp_v, cp_v, g0, g1, g2, g3, z_v, S_sh,
                  gs0, gs1, gs2, gs3):
    c = lax.axis_index("c")
    s = lax.axis_index("s")
    gb = (g0, g1, g2, g3)
    gs = (gs0, gs1, gs2, gs3)

    def _fill_z(i, _):
        j = i // (HD // L)
        g = i % (HD // L)
        z_v[j, pl.ds(g * L, L)] = jnp.zeros((L,), jnp.float32)
        return 0
    lax.fori_loop(0, ZR * (HD // L), _fill_z, 0)

    def _zero(i, _):
        pltpu.sync_copy(z_v, S_sh.at[pl.ds(s * STRIPE + i * ZR, ZR)])
        return 0
    lax.fori_loop(0, STRIPE // ZR, _zero, 0)

    delta = c * N

    for half in range(2):
        base = (s * 2 + half) * NCHH
        pltpu.sync_copy(colp_hbm.at[pl.ds(base, NCHH)], cp_v)
        pltpu.sync_copy(row_hbm.at[pl.ds(base, NCHH)], rp_v)

        def _addoff(i, _):
            j = i // (CH2 // L)
            g = i % (CH2 // L)
            rp_v[j, pl.ds(g * L, L)] = rp_v[j, pl.ds(g * L, L)] + delta
            return 0
        lax.fori_loop(0, NCHH * (CH2 // L), _addoff, 0)
        if half == 0:
            plsc.subcore_barrier()

        for k in range(NBUF):
            pltpu.async_copy(xtab_hbm.at[rp_v.at[k]], gb[k], gs[k])

        def _pipe(h, _):
            j = NBUF * h
            for k in range(NBUF):
                pltpu.make_async_copy(
                    xtab_hbm.at[rp_v.at[j + k]], gb[k], gs[k]).wait()
                pltpu.sync_copy(gb[k], S_sh.at[cp_v.at[j + k]], add=True)
                pltpu.async_copy(
                    xtab_hbm.at[rp_v.at[j + k + NBUF]], gb[k], gs[k])
            return 0
        lax.fori_loop(0, NCHH // NBUF - 1, _pipe, 0)

        for k in range(NBUF):
            j = NCHH - NBUF + k
            pltpu.make_async_copy(
                xtab_hbm.at[rp_v.at[j]], gb[k], gs[k]).wait()
            pltpu.sync_copy(gb[k], S_sh.at[cp_v.at[j]], add=True)

    plsc.subcore_barrier()
    pltpu.sync_copy(S_sh.at[pl.ds(s * STRIPE, STRIPE)],
                    out_hbm.at[pl.ds(c * SROWS + s * STRIPE, STRIPE)])


_scatter_kernel = functools.partial(
    pl.kernel,
    out_type=jax.ShapeDtypeStruct((NH * SROWS, HD), jnp.float32),
    mesh=_mesh,
    compiler_params=pltpu.CompilerParams(use_tc_tiling_on_sc=False),
    scratch_types=[
        pltpu.VMEM((NCHH, CH2), jnp.int32),
        pltpu.VMEM((NCHH, CH2), jnp.int32),
        pltpu.VMEM((CH2, HD), jnp.float32),
        pltpu.VMEM((CH2, HD), jnp.float32),
        pltpu.VMEM((CH2, HD), jnp.float32),
        pltpu.VMEM((CH2, HD), jnp.float32),
        pltpu.VMEM((ZR, HD), jnp.float32),
        pltpu.VMEM_SHARED((SROWS, HD), jnp.float32),
        pltpu.SemaphoreType.DMA,
        pltpu.SemaphoreType.DMA,
        pltpu.SemaphoreType.DMA,
        pltpu.SemaphoreType.DMA,
    ],
)(_scatter_body)


def _scale_body(x_ref, deg_ref, out_ref):
    deg = deg_ref[:, 0:1] + deg_ref[:, L:L + 1]
    dinv = jnp.where(deg > 0.0, lax.rsqrt(deg), 0.0)
    out_ref[...] = x_ref[...] * dinv


def _scale_x(x, deg2):
    BR = 1000
    grid = N // BR
    return pl.pallas_call(
        _scale_body,
        grid=(NH, grid),
        in_specs=[
            pl.BlockSpec((BR, HD), lambda h, i: (i, h)),
            pl.BlockSpec((BR, NC * L), lambda h, i: (i, 0)),
        ],
        out_specs=pl.BlockSpec((BR, HD), lambda h, i: (h * grid + i, 0)),
        out_shape=jax.ShapeDtypeStruct((NH * N, HD), jnp.float32),
    )(x, deg2)


def _xw_body(x_ref, wz_ref, wh_ref, out_ref):
    out_ref[:, :F] = jnp.dot(x_ref[...], wz_ref[...],
                             preferred_element_type=jnp.float32)
    out_ref[:, F:] = jnp.dot(x_ref[...], wh_ref[...],
                             preferred_element_type=jnp.float32)


def _xw_dense(x, wz0, wh0):
    BR = 2000
    grid = N // BR
    return pl.pallas_call(
        _xw_body,
        grid=(grid,),
        in_specs=[
            pl.BlockSpec((BR, D), lambda i: (i, 0)),
            pl.BlockSpec((D, F), lambda i: (0, 0)),
            pl.BlockSpec((D, F), lambda i: (0, 0)),
        ],
        out_specs=pl.BlockSpec((BR, 2 * F), lambda i: (i, 0)),
        out_shape=jax.ShapeDtypeStruct((N, 2 * F), jnp.float32),
    )(x, wz0, wh0)


def _fused_body(p_ref, sa_ref, sb_ref, deg_ref, wz_ref, wh_ref,
                bz_ref, bh_ref, wl_ref, bl_ref, out_ref):
    deg = deg_ref[:, 0:1] + deg_ref[:, L:L + 1]
    ndinv = jnp.where(deg > 0.0, -lax.rsqrt(deg), 0.0)
    t = jnp.concatenate([ndinv * sa_ref[...], ndinv * sb_ref[...]], axis=1)
    az = (p_ref[:, :F] + bz_ref[...]
          + jnp.dot(t, wz_ref[...], preferred_element_type=jnp.float32))
    ah = (p_ref[:, F:] + bh_ref[...]
          + jnp.dot(t, wh_ref[...], preferred_element_type=jnp.float32))
    h = jnp.maximum((1.0 - jax.nn.sigmoid(az)) * jnp.tanh(ah), 0.0)
    out_ref[...] = (jnp.sum(h * wl_ref[...], axis=1, keepdims=True)
                    + bl_ref[...])


def _fused_dense(p, sa, sb, degn, wz1, wh1, bz, bh, wlt, bl2):
    BR = 1000
    grid = N // BR
    row = lambda i: (i, 0)
    full = lambda i: (0, 0)
    return pl.pallas_call(
        _fused_body,
        grid=(grid,),
        in_specs=[
            pl.BlockSpec((BR, 2 * F), row),
            pl.BlockSpec((BR, HD), row),
            pl.BlockSpec((BR, HD), row),
            pl.BlockSpec((BR, NC * L), row),
            pl.BlockSpec((D, F), full),
            pl.BlockSpec((D, F), full),
            pl.BlockSpec((1, F), full),
            pl.BlockSpec((1, F), full),
            pl.BlockSpec((1, F), full),
            pl.BlockSpec((1, 1), full),
        ],
        out_specs=pl.BlockSpec((BR, 1), row),
        out_shape=jax.ShapeDtypeStruct((N, 1), jnp.float32),
    )(p, sa, sb, degn, wz1, wh1, bz, bh, wlt, bl2)


def kernel(x, edge_index, Wxz0, Wxz1, bxz, Whz0, Whz1, bhz, Wxr0, Wxr1, bxr,
           Whr0, Whr1, bhr, Wxh0, Wxh1, bxh, Whh0, Whh1, bhh, Wl, bl):
    row = edge_index[0]
    col = edge_index[1]
    pad = jnp.zeros((EP - E,), jnp.int32)
    row2 = jnp.concatenate([row, pad]).reshape(ROWS_ALL, CH)
    col2 = jnp.concatenate([col, pad]).reshape(ROWS_ALL, CH)

    colp2, deg2 = _deg_kernel(row2, col2)
    degn = lax.slice(deg2, (0, 0), (N, NC * L))

    xtab = _scale_x(x, degn)

    p = _xw_dense(x, Wxz0, Wxh0)

    s2 = _scatter_kernel(row2.reshape(EP // CH2, CH2),
                         colp2.reshape(EP // CH2, CH2),
                         xtab)
    sa = lax.slice(s2, (0, 0), (N, HD))
    sb = lax.slice(s2, (SROWS, 0), (SROWS + N, HD))

    return _fused_dense(p, sa, sb, degn, Wxz1, Wxh1,
                        (bxz + bhz).reshape(1, F), (bxh + bhh).reshape(1, F),
                        Wl.reshape(1, F), bl.reshape(1, 1))
